# baseline (speedup 1.0000x reference)
import jax
import jax.numpy as jnp
from jax.experimental import pallas as pl
from jax.experimental.pallas import tpu as pltpu

_BLOCK_ROWS = 10000


def _copy_body(w_ref, o_ref):
    o_ref[...] = w_ref[...]


def kernel(weight, edge_index):
    n, d = weight.shape
    return pl.pallas_call(
        _copy_body,
        out_shape=jax.ShapeDtypeStruct((n, d), weight.dtype),
        grid=(n // _BLOCK_ROWS,),
        in_specs=[pl.BlockSpec((_BLOCK_ROWS, d), lambda i: (i, 0))],
        out_specs=pl.BlockSpec((_BLOCK_ROWS, d), lambda i: (i, 0)),
        compiler_params=pltpu.CompilerParams(
            dimension_semantics=("parallel",),
        ),
    )(weight)


# speedup vs baseline: 1.0446x; 1.0446x over previous
import jax
import jax.numpy as jnp
from jax.experimental import pallas as pl
from jax.experimental.pallas import tpu as pltpu

_BLOCK_ROWS = 20000


def _copy_body(w_ref, o_ref):
    o_ref[...] = w_ref[...]


def kernel(weight, edge_index):
    n, d = weight.shape
    return pl.pallas_call(
        _copy_body,
        out_shape=jax.ShapeDtypeStruct((n, d), weight.dtype),
        grid=(n // _BLOCK_ROWS,),
        in_specs=[pl.BlockSpec((_BLOCK_ROWS, d), lambda i: (i, 0))],
        out_specs=pl.BlockSpec((_BLOCK_ROWS, d), lambda i: (i, 0)),
        compiler_params=pltpu.CompilerParams(
            dimension_semantics=("arbitrary",),
        ),
    )(weight)
